# baseline (speedup 1.0000x reference)
import functools

import jax
import jax.numpy as jnp
from jax import lax
from jax.experimental import pallas as pl
from jax.experimental.pallas import tpu as pltpu
from jax.experimental.pallas import tpu_sc as plsc

FILLER_NUM = 32
THRESHOLD = 0.5
ROWS = 128
COLS = 32768

NC = 2
NS = 16
LANES = 16
NW = NC * NS
RPW = ROWS // NW

CH = 2048
NCHUNK = COLS // CH
VPS = 8
SUB = VPS * LANES
NSUB = CH // SUB
ZN = (COLS - CH) // 4
NZDMA = (COLS - CH) // ZN

_ZSLICES = tuple((CH + z * ZN, ZN) for z in range(NZDMA))


def _sc_body(in_hbm, fp_hbm, out_hbm, zeros_v, in_v, fp_v, out_v,
             count_s, zdone_s, lsem, zsem, st_sem):
    zero16f = jnp.zeros((LANES,), jnp.float32)

    wid = lax.axis_index("s") * NC + lax.axis_index("c")
    base_row = wid * RPW

    def start_loads(r):
        row = base_row + r
        pltpu.async_copy(in_hbm.at[row, pl.ds(0, CH)], in_v.at[r], lsem)
        pltpu.async_copy(fp_hbm.at[row, pl.ds(0, CH)], fp_v.at[r], lsem)

    def wait_loads(r):
        row = base_row + r
        pltpu.make_async_copy(
            in_hbm.at[row, pl.ds(0, CH)], in_v.at[r], lsem).wait()
        pltpu.make_async_copy(
            fp_hbm.at[row, pl.ds(0, CH)], fp_v.at[r], lsem).wait()

    def wait_zeros():
        def wz(r, _):
            row = base_row + r
            for off, n in _ZSLICES:
                pltpu.make_async_copy(
                    zeros_v.at[pl.ds(0, n)],
                    out_hbm.at[row, pl.ds(off, n)], zsem).wait()
            return 0

        lax.fori_loop(0, RPW, wz, 0)

    def zinit(i, _):
        for u in range(16):
            zeros_v[pl.ds((i * 16 + u) * LANES, LANES)] = zero16f
        return 0

    lax.fori_loop(0, ZN // (16 * LANES), zinit, 0)

    def start_zeros(r, _):
        row = base_row + r
        for off, n in _ZSLICES:
            pltpu.async_copy(zeros_v.at[pl.ds(0, n)],
                             out_hbm.at[row, pl.ds(off, n)], zsem)
        return 0

    lax.fori_loop(0, RPW, start_zeros, 0)
    zdone_s[0] = jnp.int32(0)

    def prefetch(r, _):
        start_loads(r)
        return 0

    lax.fori_loop(0, RPW, prefetch, 0)

    def drain_loads(r, _):
        wait_loads(r)
        return 0

    lax.fori_loop(0, RPW, drain_loads, 0)

    def process_chunk(r):
        def sub(s, _):
            p = count_s[0] < jnp.int32(FILLER_NUM)

            @pl.when(p)
            def _():
                cnt0 = count_s[0]
                ms = []
                pcv = jnp.zeros((LANES,), jnp.int32)
                for k in range(VPS):
                    off = s * SUB + k * LANES
                    a = in_v[r, pl.ds(off, LANES)]
                    f = fp_v[r, pl.ds(off, LANES)]
                    m = (f > THRESHOLD) & (a > 0.0)
                    ms.append(m)
                    pcv = pcv + plsc.all_reduce_population_count(m)
                newcnt = cnt0 + jnp.max(pcv)
                crosses = newcnt > jnp.int32(FILLER_NUM)

                @pl.when(jnp.logical_not(crosses))
                def _():
                    for k in range(VPS):
                        out_v[r, pl.ds(s * SUB + k * LANES, LANES)] = (
                            jnp.where(ms[k], jnp.float32(1.0),
                                      jnp.float32(0.0)))
                    count_s[0] = newcnt

                @pl.when(crosses)
                def _():
                    cntv = cnt0 + jnp.zeros((LANES,), jnp.int32)
                    for k in range(VPS):
                        csum = plsc.cumsum(ms[k].astype(jnp.int32))
                        keep = (cntv + csum) <= jnp.int32(FILLER_NUM)
                        out_v[r, pl.ds(s * SUB + k * LANES, LANES)] = (
                            jnp.where(keep & ms[k], jnp.float32(1.0),
                                      jnp.float32(0.0)))
                        cntv = cntv + plsc.all_reduce_population_count(
                            ms[k])
                    count_s[0] = jnp.max(cntv)

            @pl.when(jnp.logical_not(p))
            def _():
                for k in range(VPS):
                    out_v[r, pl.ds(s * SUB + k * LANES, LANES)] = zero16f

            return 0

        lax.fori_loop(0, NSUB, sub, 0)

    def do_row(r, _):
        row = base_row + r
        count_s[0] = jnp.int32(0)
        process_chunk(r)
        pltpu.async_copy(out_v.at[r], out_hbm.at[row, pl.ds(0, CH)], st_sem)

        @pl.when(count_s[0] < jnp.int32(FILLER_NUM))
        def _():
            @pl.when(zdone_s[0] == jnp.int32(0))
            def _():
                wait_zeros()
                zdone_s[0] = jnp.int32(1)

            def rare_chunk(c, _):
                @pl.when(count_s[0] < jnp.int32(FILLER_NUM))
                def _():
                    pltpu.sync_copy(
                        in_hbm.at[row, pl.ds(c * CH, CH)], in_v.at[r])
                    pltpu.sync_copy(
                        fp_hbm.at[row, pl.ds(c * CH, CH)], fp_v.at[r])
                    process_chunk(r)
                    pltpu.sync_copy(
                        out_v.at[r], out_hbm.at[row, pl.ds(c * CH, CH)])

                return 0

            lax.fori_loop(1, NCHUNK, rare_chunk, 0)

        return 0

    lax.fori_loop(0, RPW, do_row, 0)

    def drain_store(r, _):
        row = base_row + r
        pltpu.make_async_copy(
            out_v.at[r], out_hbm.at[row, pl.ds(0, CH)], st_sem).wait()
        return 0

    lax.fori_loop(0, RPW, drain_store, 0)

    @pl.when(zdone_s[0] == jnp.int32(0))
    def _():
        wait_zeros()


_sc_kernel = functools.partial(
    pl.kernel,
    out_type=jax.ShapeDtypeStruct((ROWS, COLS), jnp.float32),
    mesh=plsc.VectorSubcoreMesh(core_axis_name="c", subcore_axis_name="s"),
    compiler_params=pltpu.CompilerParams(needs_layout_passes=False),
    scratch_types=[
        pltpu.VMEM((ZN,), jnp.float32),
        pltpu.VMEM((RPW, CH), jnp.float32),
        pltpu.VMEM((RPW, CH), jnp.float32),
        pltpu.VMEM((RPW, CH), jnp.float32),
        pltpu.SMEM((1,), jnp.int32),
        pltpu.SMEM((1,), jnp.int32),
        pltpu.SemaphoreType.DMA,
        pltpu.SemaphoreType.DMA,
        pltpu.SemaphoreType.DMA,
    ],
)(_sc_body)


def kernel(input, fake_param):
    return _sc_kernel(input, fake_param)


# speedup vs baseline: 1.0038x; 1.0038x over previous
import functools

import jax
import jax.numpy as jnp
from jax import lax
from jax.experimental import pallas as pl
from jax.experimental.pallas import tpu as pltpu
from jax.experimental.pallas import tpu_sc as plsc

FILLER_NUM = 32
THRESHOLD = 0.5
ROWS = 128
COLS = 32768

NC = 2
NS = 16
LANES = 16
NW = NC * NS
RPW = ROWS // NW

CH = 2048
NCHUNK = COLS // CH
VPS = 8
SUB = VPS * LANES
NSUB = CH // SUB
ZN = (COLS - CH) // 4
NZDMA = (COLS - CH) // ZN

_ZSLICES = tuple((CH + z * ZN, ZN) for z in range(NZDMA))


def _sc_body(in_hbm, fp_hbm, out_hbm, zeros_v, in_v, fp_v, out_v,
             count_s, zdone_s, lsem, zsem, st_sem):
    zero16f = jnp.zeros((LANES,), jnp.float32)

    wid = lax.axis_index("s") * NC + lax.axis_index("c")
    base_row = wid * RPW

    def start_loads(r):
        row = base_row + r
        pltpu.async_copy(in_hbm.at[row, pl.ds(0, CH)], in_v.at[r], lsem)
        pltpu.async_copy(fp_hbm.at[row, pl.ds(0, CH)], fp_v.at[r], lsem)

    def wait_loads(r):
        row = base_row + r
        pltpu.make_async_copy(
            in_hbm.at[row, pl.ds(0, CH)], in_v.at[r], lsem).wait()
        pltpu.make_async_copy(
            fp_hbm.at[row, pl.ds(0, CH)], fp_v.at[r], lsem).wait()

    band = wid % 16
    chalf = wid // 16

    def zero_blocks():
        return [(band * 8, CH + (chalf * 15 + j) * 1024) for j in range(15)]

    def wait_zeros():
        for r0, c0 in zero_blocks():
            pltpu.make_async_copy(
                zeros_v, out_hbm.at[pl.ds(r0, 8), pl.ds(c0, 1024)],
                zsem).wait()

    def zinit(i, _):
        for u in range(16):
            j = i * 16 + u
            zeros_v[j // 64, pl.ds((j % 64) * LANES, LANES)] = zero16f
        return 0

    lax.fori_loop(0, (8 * 1024) // (16 * LANES), zinit, 0)

    for r0, c0 in zero_blocks():
        pltpu.async_copy(
            zeros_v, out_hbm.at[pl.ds(r0, 8), pl.ds(c0, 1024)], zsem)
    zdone_s[0] = jnp.int32(0)

    def prefetch(r, _):
        start_loads(r)
        return 0

    lax.fori_loop(0, RPW, prefetch, 0)

    def drain_loads(r, _):
        wait_loads(r)
        return 0

    lax.fori_loop(0, RPW, drain_loads, 0)

    def process_chunk(r):
        def sub(s, _):
            p = count_s[0] < jnp.int32(FILLER_NUM)

            @pl.when(p)
            def _():
                cnt0 = count_s[0]
                ms = []
                pcv = jnp.zeros((LANES,), jnp.int32)
                for k in range(VPS):
                    off = s * SUB + k * LANES
                    a = in_v[r, pl.ds(off, LANES)]
                    f = fp_v[r, pl.ds(off, LANES)]
                    m = (f > THRESHOLD) & (a > 0.0)
                    ms.append(m)
                    pcv = pcv + plsc.all_reduce_population_count(m)
                newcnt = cnt0 + jnp.max(pcv)
                crosses = newcnt > jnp.int32(FILLER_NUM)

                @pl.when(jnp.logical_not(crosses))
                def _():
                    for k in range(VPS):
                        out_v[r, pl.ds(s * SUB + k * LANES, LANES)] = (
                            jnp.where(ms[k], jnp.float32(1.0),
                                      jnp.float32(0.0)))
                    count_s[0] = newcnt

                @pl.when(crosses)
                def _():
                    cntv = cnt0 + jnp.zeros((LANES,), jnp.int32)
                    for k in range(VPS):
                        csum = plsc.cumsum(ms[k].astype(jnp.int32))
                        keep = (cntv + csum) <= jnp.int32(FILLER_NUM)
                        out_v[r, pl.ds(s * SUB + k * LANES, LANES)] = (
                            jnp.where(keep & ms[k], jnp.float32(1.0),
                                      jnp.float32(0.0)))
                        cntv = cntv + plsc.all_reduce_population_count(
                            ms[k])
                    count_s[0] = jnp.max(cntv)

            @pl.when(jnp.logical_not(p))
            def _():
                for k in range(VPS):
                    out_v[r, pl.ds(s * SUB + k * LANES, LANES)] = zero16f

            return 0

        lax.fori_loop(0, NSUB, sub, 0)

    def do_row(r, _):
        row = base_row + r
        count_s[0] = jnp.int32(0)
        process_chunk(r)
        pltpu.async_copy(out_v.at[r], out_hbm.at[row, pl.ds(0, CH)], st_sem)

        @pl.when(count_s[0] < jnp.int32(FILLER_NUM))
        def _():
            @pl.when(zdone_s[0] == jnp.int32(0))
            def _():
                wait_zeros()
                zdone_s[0] = jnp.int32(1)

            def rare_chunk(c, _):
                @pl.when(count_s[0] < jnp.int32(FILLER_NUM))
                def _():
                    pltpu.sync_copy(
                        in_hbm.at[row, pl.ds(c * CH, CH)], in_v.at[r])
                    pltpu.sync_copy(
                        fp_hbm.at[row, pl.ds(c * CH, CH)], fp_v.at[r])
                    process_chunk(r)
                    pltpu.sync_copy(
                        out_v.at[r], out_hbm.at[row, pl.ds(c * CH, CH)])

                return 0

            lax.fori_loop(1, NCHUNK, rare_chunk, 0)

        return 0

    lax.fori_loop(0, RPW, do_row, 0)

    def drain_store(r, _):
        row = base_row + r
        pltpu.make_async_copy(
            out_v.at[r], out_hbm.at[row, pl.ds(0, CH)], st_sem).wait()
        return 0

    lax.fori_loop(0, RPW, drain_store, 0)

    @pl.when(zdone_s[0] == jnp.int32(0))
    def _():
        wait_zeros()


_sc_kernel = functools.partial(
    pl.kernel,
    out_type=jax.ShapeDtypeStruct((ROWS, COLS), jnp.float32),
    mesh=plsc.VectorSubcoreMesh(core_axis_name="c", subcore_axis_name="s"),
    compiler_params=pltpu.CompilerParams(needs_layout_passes=False),
    scratch_types=[
        pltpu.VMEM((8, 1024), jnp.float32),
        pltpu.VMEM((RPW, CH), jnp.float32),
        pltpu.VMEM((RPW, CH), jnp.float32),
        pltpu.VMEM((RPW, CH), jnp.float32),
        pltpu.SMEM((1,), jnp.int32),
        pltpu.SMEM((1,), jnp.int32),
        pltpu.SemaphoreType.DMA,
        pltpu.SemaphoreType.DMA,
        pltpu.SemaphoreType.DMA,
    ],
)(_sc_body)


def kernel(input, fake_param):
    return _sc_kernel(input, fake_param)
